# Initial kernel scaffold; baseline (speedup 1.0000x reference)
import jax
import jax.numpy as jnp
from jax.experimental import pallas as pl


def kernel(input_ids, scores_ids, words_emb, scores_emb):
    raise NotImplementedError("write your pallas kernel here")


# speedup vs baseline: 1.6635x; 1.6635x over previous
import functools

import jax
import jax.numpy as jnp
from jax import lax
from jax.experimental import pallas as pl
from jax.experimental.pallas import tpu as pltpu
from jax.experimental.pallas import tpu_sc as plsc

VOCAB = 1000000
BINS = 100
D = 64
N = 4096 * 200
NC, NS = 2, 16
NW = NC * NS
PER_W = N // NW
G = 128
NG = PER_W // G


def _body(wids, sids, wtab, stab, out, widx_v, sidx_v, rows_v, srows_v):
    wid = lax.axis_index("s") * NC + lax.axis_index("c")
    pltpu.sync_copy(wids.at[wid], widx_v)
    pltpu.sync_copy(sids.at[wid], sidx_v)
    base0 = wid * PER_W

    @pl.loop(0, NG)
    def _group(g):
        pltpu.sync_copy(wtab.at[widx_v.at[g]], rows_v)
        pltpu.sync_copy(stab.at[sidx_v.at[g]], srows_v)

        @pl.loop(0, G, unroll=4)
        def _row(i):
            for j in range(D // 16):
                sl = pl.ds(j * 16, 16)
                rows_v[i, sl] = rows_v[i, sl] + srows_v[i, sl]

        pltpu.sync_copy(rows_v, out.at[pl.ds(base0 + g * G, G)])


@jax.jit
def _sc_embed(wids, sids, wtab, stab):
    kern = pl.kernel(
        _body,
        out_type=jax.ShapeDtypeStruct((N, D), jnp.float32),
        mesh=plsc.VectorSubcoreMesh(core_axis_name="c", subcore_axis_name="s"),
        compiler_params=pltpu.CompilerParams(use_tc_tiling_on_sc=False),
        scratch_types=[
            pltpu.VMEM((NG, G), jnp.int32),
            pltpu.VMEM((NG, G), jnp.int32),
            pltpu.VMEM((G, D), jnp.float32),
            pltpu.VMEM((G, D), jnp.float32),
        ],
    )
    return kern(wids, sids, wtab, stab)


def kernel(input_ids, scores_ids, words_emb, scores_emb):
    wids = input_ids.reshape(NW, NG, G).astype(jnp.int32)
    sids = scores_ids.reshape(NW, NG, G).astype(jnp.int32)
    out = _sc_embed(wids, sids, words_emb, scores_emb)
    return out.reshape(input_ids.shape + (D,))


# speedup vs baseline: 1.9624x; 1.1797x over previous
import jax
import jax.numpy as jnp
from jax import lax
from jax.experimental import pallas as pl
from jax.experimental.pallas import tpu as pltpu
from jax.experimental.pallas import tpu_sc as plsc

VOCAB = 1000000
BINS = 100
D = 64
N = 4096 * 200
NC, NS = 2, 16
NW = NC * NS
PER_W = N // NW
G = 128
NG = PER_W // G
NB = 2


def _body(wids, sids, wtab, stab, out, widx_v, sidx_v, rows_v, srows_v,
          obuf_v, gsemw, gsems, ssem):
    wid = lax.axis_index("s") * NC + lax.axis_index("c")
    pltpu.sync_copy(wids.at[wid], widx_v)
    pltpu.sync_copy(sids.at[wid], sidx_v)
    base0 = wid * PER_W

    def start_gathers(g, b):
        pltpu.make_async_copy(
            wtab.at[widx_v.at[g]], rows_v.at[b], gsemw.at[b]).start()
        pltpu.make_async_copy(
            stab.at[sidx_v.at[g]], srows_v.at[b], gsems.at[b]).start()

    def wait_gathers(g, b):
        pltpu.make_async_copy(
            wtab.at[widx_v.at[g]], rows_v.at[b], gsemw.at[b]).wait()
        pltpu.make_async_copy(
            stab.at[sidx_v.at[g]], srows_v.at[b], gsems.at[b]).wait()

    def scatter_desc(g, b):
        return pltpu.make_async_copy(
            obuf_v.at[b], out.at[pl.ds(base0 + g * G, G)], ssem.at[b])

    for b in range(NB):
        start_gathers(b, b)

    @pl.loop(0, NG, step=NB)
    def _group(g0):
        for b in range(NB):
            g = g0 + b
            wait_gathers(g, b)

            @pl.when(g0 >= NB)
            def _():
                scatter_desc(g - NB, b).wait()

            @pl.loop(0, G, unroll=8)
            def _row(i):
                for j in range(D // 16):
                    sl = pl.ds(j * 16, 16)
                    obuf_v[b, i, sl] = rows_v[b, i, sl] + srows_v[b, i, sl]

            scatter_desc(g, b).start()

            @pl.when(g + NB < NG)
            def _():
                start_gathers(g + NB, b)

    for b in range(NB):
        scatter_desc(NG - NB + b, b).wait()


@jax.jit
def _sc_embed(wids, sids, wtab, stab):
    kern = pl.kernel(
        _body,
        out_type=jax.ShapeDtypeStruct((N, D), jnp.float32),
        mesh=plsc.VectorSubcoreMesh(core_axis_name="c", subcore_axis_name="s"),
        compiler_params=pltpu.CompilerParams(use_tc_tiling_on_sc=False),
        scratch_types=[
            pltpu.VMEM((NG, G), jnp.int32),
            pltpu.VMEM((NG, G), jnp.int32),
            pltpu.VMEM((NB, G, D), jnp.float32),
            pltpu.VMEM((NB, G, D), jnp.float32),
            pltpu.VMEM((NB, G, D), jnp.float32),
            pltpu.SemaphoreType.DMA((NB,)),
            pltpu.SemaphoreType.DMA((NB,)),
            pltpu.SemaphoreType.DMA((NB,)),
        ],
    )
    return kern(wids, sids, wtab, stab)


def kernel(input_ids, scores_ids, words_emb, scores_emb):
    wids = input_ids.reshape(NW, NG, G).astype(jnp.int32)
    sids = scores_ids.reshape(NW, NG, G).astype(jnp.int32)
    out = _sc_embed(wids, sids, words_emb, scores_emb)
    return out.reshape(input_ids.shape + (D,))
